# Initial kernel scaffold; baseline (speedup 1.0000x reference)
import jax
import jax.numpy as jnp
from jax.experimental import pallas as pl


def kernel(x, edge_index, batch, Wl1, bl1, Wr1, w1, Wl2, bl2, Wr2, w2, Wl3, bl3, Wr3, w3, lin1_W, lin1_b, lin2_W, lin2_b, lin3_W, lin3_b):
    raise NotImplementedError("write your pallas kernel here")


# speedup vs baseline: 12.9937x; 12.9937x over previous
import functools
import math

import jax
import jax.numpy as jnp
from jax import lax
from jax.experimental import pallas as pl
from jax.experimental.pallas import tpu as pltpu
from jax.experimental.pallas import tpu_sc as plsc

N = 10000
NP = 10240
D = 128
HD = 64
E = 320000
NC = 2
NS = 16
CH = 128
NCHK = 157
EPT = CH * NCHK
EPAD = EPT * NS
DUMP = N + 100
RPT = NP // NS

def _sc_agg_body(src_hbm, dst_hbm, xt_hbm, keep_hbm, agg_out, cnt_out,
                 src_v, dst_v, keep_v, rows_v, cbuf_v, zf_v, zc_v,
                 agg_sm, cnt_sm, sem):
    c = lax.axis_index("c")
    s = lax.axis_index("s")

    z16 = jnp.zeros((16,), jnp.float32)
    lane = lax.iota(jnp.int32, 16)
    col0 = jnp.zeros((16,), jnp.int32)

    def _zrow(i, carry):
        for kk in range(HD // 16):
            zf_v[i, pl.ds(kk * 16, 16)] = z16
        return carry
    lax.fori_loop(0, CH, _zrow, 0)

    def _zcrow(i, carry):
        zc_v[i, :] = z16
        return carry
    lax.fori_loop(0, RPT, _zcrow, 0)

    def _zcb(i, carry):
        cbuf_v[i, :] = z16
        return carry
    lax.fori_loop(0, CH, _zcb, 0)

    for b in range(RPT // CH):
        pltpu.sync_copy(zf_v, agg_sm.at[pl.ds(s * RPT + b * CH, CH)])
    pltpu.sync_copy(zc_v, cnt_sm.at[pl.ds(s * RPT, RPT)])

    pltpu.sync_copy(src_hbm.at[s], src_v)
    pltpu.sync_copy(dst_hbm.at[s], dst_v)
    pltpu.sync_copy(keep_hbm, keep_v)

    plsc.subcore_barrier()

    half = NCHK // 2 + 1

    def _chunk(j, carry):
        pltpu.async_copy(xt_hbm.at[c].at[src_v.at[j]], rows_v, sem).wait()
        pltpu.sync_copy(rows_v, agg_sm.at[dst_v.at[j]], add=True)

        @pl.when((j < half) == (c == 0))
        def _():
            for i in range(CH // 16):
                s16 = src_v[j, pl.ds(i * 16, 16)]
                k16 = plsc.load_gather(keep_v, [s16])
                plsc.store_scatter(cbuf_v, [i * 16 + lane, col0], k16)
            pltpu.sync_copy(cbuf_v, cnt_sm.at[dst_v.at[j]], add=True)
        return carry

    lax.fori_loop(0, NCHK, _chunk, 0)

    plsc.subcore_barrier()

    pltpu.sync_copy(agg_sm.at[pl.ds(s * RPT, RPT)],
                    agg_out.at[c, pl.ds(s * RPT, RPT)])
    pltpu.sync_copy(cnt_sm.at[pl.ds(s * RPT, RPT)],
                    cnt_out.at[c, pl.ds(s * RPT, RPT)])


@functools.lru_cache(maxsize=1)
def _get_sc_agg():
    mesh = plsc.VectorSubcoreMesh(core_axis_name="c", subcore_axis_name="s")
    return pl.kernel(
        _sc_agg_body,
        out_type=(
            jax.ShapeDtypeStruct((NC, NP, HD), jnp.float32),
            jax.ShapeDtypeStruct((NC, NP, 16), jnp.float32),
        ),
        mesh=mesh,
        compiler_params=pltpu.CompilerParams(needs_layout_passes=False,
                                             use_tc_tiling_on_sc=False),
        scratch_types=[
            pltpu.VMEM((NCHK, CH), jnp.int32),
            pltpu.VMEM((NCHK, CH), jnp.int32),
            pltpu.VMEM((NP,), jnp.float32),
            pltpu.VMEM((CH, HD), jnp.float32),
            pltpu.VMEM((CH, 16), jnp.float32),
            pltpu.VMEM((CH, HD), jnp.float32),
            pltpu.VMEM((RPT, 16), jnp.float32),
            pltpu.VMEM_SHARED((NP, HD), jnp.float32),
            pltpu.VMEM_SHARED((NP, 16), jnp.float32),
            pltpu.SemaphoreType.DMA,
        ],
    )


BS = 1024


def _tc_dense_body(aggp_ref, cntp_ref, xp_ref, Wl_ref, bl_ref, Wr_ref, w_ref,
                   h_ref, sc_ref):
    agg = jnp.concatenate([aggp_ref[0], aggp_ref[1]], axis=1)
    xp = jnp.concatenate([xp_ref[0], xp_ref[1]], axis=1)
    cnt = cntp_ref[0, :, 0:1] + cntp_ref[1, :, 0:1]
    mean = agg / jnp.maximum(cnt, 1.0)
    h = mean @ Wl_ref[...] + bl_ref[...] + xp @ Wr_ref[...]
    h = jnp.maximum(h, 0.0)
    h_ref[...] = h
    wv = w_ref[...]
    wn = wv * lax.rsqrt(jnp.sum(wv * wv))
    sc_ref[...] = jnp.tanh(jnp.sum(h * wn, axis=1, keepdims=True))


_tc_dense = pl.pallas_call(
    _tc_dense_body,
    grid=(NP // BS,),
    in_specs=[
        pl.BlockSpec((NC, BS, HD), lambda i: (0, i, 0)),
        pl.BlockSpec((NC, BS, 16), lambda i: (0, i, 0)),
        pl.BlockSpec((NC, BS, HD), lambda i: (0, i, 0)),
        pl.BlockSpec((D, D), lambda i: (0, 0)),
        pl.BlockSpec((1, D), lambda i: (0, 0)),
        pl.BlockSpec((D, D), lambda i: (0, 0)),
        pl.BlockSpec((1, D), lambda i: (0, 0)),
    ],
    out_specs=(
        pl.BlockSpec((BS, D), lambda i: (i, 0)),
        pl.BlockSpec((BS, 1), lambda i: (i, 0)),
    ),
    out_shape=(
        jax.ShapeDtypeStruct((NP, D), jnp.float32),
        jax.ShapeDtypeStruct((NP, 1), jnp.float32),
    ),
)


def _tc_layer_body(k, h_ref, sc_ref, keep_ref, pnew_ref, keepn_ref, ro_ref):
    h = h_ref[...]
    sc = sc_ref[...]
    keep_prev = keep_ref[...] > 0.0
    neg = jnp.float32(-jnp.inf)
    sm = jnp.where(keep_prev, sc, neg)

    u = lax.bitcast_convert_type(sm, jnp.uint32)
    flip = jnp.where(u >> 31 != 0,
                     jnp.uint32(0xFFFFFFFF), jnp.uint32(0x80000000))
    uu = u ^ flip

    def _tb(i, T):
        cand = T | (jnp.uint32(1) << (31 - i))
        nge = jnp.sum(jnp.where(uu >= cand, 1, 0))
        return jnp.where(nge >= k, cand, T)
    T = lax.fori_loop(0, 32, _tb, jnp.uint32(0))

    ngt = jnp.sum(jnp.where(uu > T, 1, 0))
    need = k - ngt
    ties = uu == T
    ridx = lax.broadcasted_iota(jnp.int32, (NP, 1), 0)

    def _rb(i, R):
        cand = R | (1 << (13 - i))
        nt = jnp.sum(jnp.where(ties & (ridx < cand), 1, 0))
        return jnp.where(nt <= need, cand, R)
    rcut = lax.fori_loop(0, 14, _rb, jnp.int32(0))

    keepn = (uu > T) | (ties & (ridx < rcut))
    ph = h * sc
    pn = jnp.where(keepn, ph, 0.0)

    pnew_ref[0] = pn[:, :HD]
    pnew_ref[1] = pn[:, HD:]
    keepn_ref[...] = keepn.astype(jnp.float32)
    mx = jnp.max(jnp.where(keepn, ph, neg), axis=0, keepdims=True)
    mn = jnp.sum(pn, axis=0, keepdims=True) * (1.0 / k)
    ro_ref[...] = jnp.concatenate([mx, mn], axis=1)


def _tc_layer(k):
    return pl.pallas_call(
        functools.partial(_tc_layer_body, k),
        out_shape=(
            jax.ShapeDtypeStruct((NC, NP, HD), jnp.float32),
            jax.ShapeDtypeStruct((NP, 1), jnp.float32),
            jax.ShapeDtypeStruct((1, 2 * D), jnp.float32),
        ),
        compiler_params=pltpu.CompilerParams(
            vmem_limit_bytes=120 * 1024 * 1024),
    )


def _tc_mlp_body(x1_ref, x2_ref, x3_ref, W1_ref, b1_ref, W2_ref, b2_ref,
                 w3_ref, b3_ref, out_ref):
    hsum = x1_ref[...] + x2_ref[...] + x3_ref[...]
    h = jnp.maximum(hsum @ W1_ref[...] + b1_ref[...], 0.0)
    h = jnp.maximum(h @ W2_ref[...] + b2_ref[...], 0.0)
    o = jnp.sum(h * w3_ref[...], axis=1, keepdims=True) + b3_ref[...]
    out_ref[...] = 1.0 / (1.0 + jnp.exp(-o))


_tc_mlp = pl.pallas_call(
    _tc_mlp_body,
    out_shape=jax.ShapeDtypeStruct((1, 1), jnp.float32),
)


def kernel(x, edge_index, batch, Wl1, bl1, Wr1, w1, Wl2, bl2, Wr2, w2,
           Wl3, bl3, Wr3, w3, lin1_W, lin1_b, lin2_W, lin2_b, lin3_W, lin3_b):
    del batch

    x2d = jnp.reshape(x, (N, D)).astype(jnp.float32)
    xfull = jnp.concatenate([x2d, jnp.zeros((NP - N, D), jnp.float32)], axis=0)
    xp = jnp.transpose(xfull.reshape(NP, NC, HD), (1, 0, 2))

    src = edge_index[0].astype(jnp.int32)
    dst = edge_index[1].astype(jnp.int32)
    pad = EPAD - E
    srcp = jnp.concatenate([src, jnp.zeros((pad,), jnp.int32)])
    dstp = jnp.concatenate([dst, jnp.full((pad,), DUMP, jnp.int32)])
    srcp = srcp.reshape(NS, NCHK, CH)
    dstp = dstp.reshape(NS, NCHK, CH)

    keep = jnp.concatenate([jnp.ones((N,), jnp.float32),
                            jnp.zeros((NP - N,), jnp.float32)])

    ks = [math.ceil(0.8 * N)]
    ks.append(math.ceil(0.8 * ks[0]))
    ks.append(math.ceil(0.8 * ks[1]))

    params = [(Wl1, bl1, Wr1, w1), (Wl2, bl2, Wr2, w2), (Wl3, bl3, Wr3, w3)]
    ros = []
    for li in range(3):
        Wl, bl, Wr, w = params[li]
        aggp, cntp = _get_sc_agg()(srcp, dstp, xp, keep)
        h, sc = _tc_dense(aggp, cntp, xp,
                          Wl, bl.reshape(1, D), Wr, w.reshape(1, D))
        xp, keep2d, ro = _tc_layer(ks[li])(h, sc, keep.reshape(NP, 1))
        keep = keep2d.reshape(NP)
        ros.append(ro)

    out = _tc_mlp(ros[0], ros[1], ros[2],
                  lin1_W, lin1_b.reshape(1, 128),
                  lin2_W, lin2_b.reshape(1, 64),
                  lin3_W.reshape(1, 64), lin3_b.reshape(1, 1))
    return out.reshape(1)
